# baseline (speedup 1.0000x reference)
import functools

import jax
import jax.numpy as jnp
from jax import lax
from jax.experimental import pallas as pl
from jax.experimental.pallas import tpu as pltpu
from jax.experimental.pallas import tpu_sc as plsc

N_NODES = 10000
D_IN = 256
D_OUT = 256

NC = 2
NT = 16
G = 128
HALF = 128
NP = 10112
SCRAP = 10016
GRP = 8
ROWS_PT = NP // NT


def _sc_aggregate(ep, xcat, src2, dst2):
    ept = ep // NT
    chunks = ept // G
    ngroups = chunks // GRP

    mesh = plsc.VectorSubcoreMesh(core_axis_name="c", subcore_axis_name="s")

    @functools.partial(
        pl.kernel,
        out_type=jax.ShapeDtypeStruct((NC, NP, HALF), jnp.float32),
        mesh=mesh,
        scratch_types=[
            pltpu.VMEM((ept,), jnp.int32),
            [pltpu.VMEM((GRP, G), jnp.int32)] * 2,
            [pltpu.VMEM((G, HALF), jnp.float32)] * 2,
            pltpu.VMEM_SHARED((NP, HALF), jnp.float32),
            [pltpu.SemaphoreType.DMA] * 2,
            [pltpu.SemaphoreType.DMA] * 2,
            [pltpu.SemaphoreType.DMA] * 2,
            pltpu.SemaphoreType.DMA,
        ],
    )
    def kernel_fn(xcat_hbm, src_hbm, dst_hbm, out_hbm,
                  src_v, dring, bufs, acc,
                  gsem, ssem, dsem, sem_i):
        c = lax.axis_index("c")
        t = lax.axis_index("s")

        cp_src = pltpu.async_copy(
            src_hbm.at[c, pl.ds(t * ept, ept)], src_v, sem_i)
        pltpu.async_copy(
            dst_hbm.at[pl.ds(t * chunks, GRP)], dring[0], dsem[0])

        z = jnp.zeros((16,), jnp.float32)

        @pl.loop(0, G)
        def _(r):
            row = bufs[0].at[r]
            for qq in range(HALF // 16):
                row[pl.ds(qq * 16, 16)] = z

        for k in range(ROWS_PT // G):
            pltpu.sync_copy(bufs[0], acc.at[pl.ds(t * ROWS_PT + k * G, G)])
        rem = ROWS_PT % G
        if rem:
            pltpu.sync_copy(
                bufs[0].at[pl.ds(0, rem)],
                acc.at[pl.ds(t * ROWS_PT + (ROWS_PT // G) * G, rem)])

        cp_src.wait()
        plsc.subcore_barrier()

        pltpu.async_copy(
            xcat_hbm.at[src_v.at[pl.ds(0, G)]], bufs[0], gsem[0])

        @pl.loop(0, ngroups // 2)
        def _(gi):
            for gg in range(2):
                for u in range(GRP):
                    j = (gi * 2 + gg) * GRP + u
                    k = (u + gg * GRP) % 2
                    kn = (k + 1) % 2

                    @pl.when(j >= 1)
                    def _():
                        pltpu.make_async_copy(
                            bufs[kn], acc.at[dring[gg].at[0]],
                            ssem[kn]).wait()

                    if u == 0:
                        pltpu.make_async_copy(
                            dst_hbm.at[pl.ds(0, GRP)], dring[gg],
                            dsem[gg]).wait()
                        g_cur = gi * 2 + gg

                        @pl.when(g_cur + 1 < ngroups)
                        def _():
                            pltpu.async_copy(
                                dst_hbm.at[
                                    pl.ds(t * chunks + (g_cur + 1) * GRP,
                                          GRP)],
                                dring[1 - gg], dsem[1 - gg])

                    @pl.when(j + 1 < chunks)
                    def _():
                        for q in range(4):
                            pltpu.async_copy(
                                xcat_hbm.at[
                                    src_v.at[pl.ds((j + 1) * G + q * 32, 32)]],
                                bufs[kn].at[pl.ds(q * 32, 32)], gsem[kn])

                    for q in range(4):
                        pltpu.make_async_copy(
                            xcat_hbm.at[src_v.at[pl.ds(j * G + q * 32, 32)]],
                            bufs[k].at[pl.ds(q * 32, 32)], gsem[k]).wait()
                    pltpu.async_copy(bufs[k], acc.at[dring[gg].at[u]],
                                     ssem[k], add=True)

        pltpu.make_async_copy(
            bufs[(chunks - 1) % 2], acc.at[dring[1].at[GRP - 1]],
            ssem[(chunks - 1) % 2]).wait()

        plsc.subcore_barrier()

        pltpu.sync_copy(
            acc.at[pl.ds(t * ROWS_PT, ROWS_PT)],
            out_hbm.at[c, pl.ds(t * ROWS_PT, ROWS_PT)],
        )

    return kernel_fn(xcat, src2, dst2)


def _tc_linear(hpair, wt, b2):
    bm = 400
    grid = (N_NODES // bm,)

    def body(hl_ref, hr_ref, wt_ref, b_ref, o_ref):
        acc = jnp.dot(hl_ref[0], wt_ref[:HALF, :],
                      preferred_element_type=jnp.float32)
        acc = acc + jnp.dot(hr_ref[0], wt_ref[HALF:, :],
                            preferred_element_type=jnp.float32)
        o_ref[...] = jnp.maximum(acc + b_ref[...], 0.0)

    return pl.pallas_call(
        body,
        grid=grid,
        in_specs=[
            pl.BlockSpec((1, bm, HALF), lambda i: (0, i, 0)),
            pl.BlockSpec((1, bm, HALF), lambda i: (1, i, 0)),
            pl.BlockSpec((D_IN, D_OUT), lambda i: (0, 0)),
            pl.BlockSpec((1, D_OUT), lambda i: (0, 0)),
        ],
        out_specs=pl.BlockSpec((bm, D_OUT), lambda i: (i, 0)),
        out_shape=jax.ShapeDtypeStruct((N_NODES, D_OUT), jnp.float32),
    )(hpair, hpair, wt, b2)


@jax.jit
def kernel(x, edge_index, W, b):
    e = edge_index.shape[1]
    quantum = NT * G * 2 * GRP
    ep = ((e + quantum - 1) // quantum) * quantum

    src = edge_index[0]
    dst = edge_index[1]
    src_p = jnp.concatenate([src, jnp.zeros((ep - e,), jnp.int32)])
    dst_p = jnp.concatenate(
        [dst, jnp.full((ep - e,), SCRAP, jnp.int32)])
    src2 = jnp.stack([2 * src_p, 2 * src_p + 1])
    dst2 = dst_p.reshape(ep // G, G)
    xcat = x.reshape(2 * N_NODES, HALF)

    hpair = _sc_aggregate(ep, xcat, src2, dst2)

    return _tc_linear(hpair, W.T, b.reshape(1, D_OUT))
